# baseline (speedup 1.0000x reference)
import functools

import jax
import jax.numpy as jnp
from jax import lax
from jax.experimental import pallas as pl
from jax.experimental.pallas import tpu as pltpu
from jax.experimental.pallas import tpu_sc as plsc

NC = 2
NS = 16
NW = NC * NS

ROWS = 16384
COLS = 32
TOTAL = ROWS * COLS
PER_W = TOTAL // NW
CHUNK = 128
NCHUNK = PER_W // CHUNK
LANES = 16


def _body(in_hbm, idx_hbm, out_hbm, addr_v, val_v, sem):
    w = lax.axis_index("s") * NC + lax.axis_index("c")

    pltpu.sync_copy(idx_hbm.at[w], addr_v)

    jlane = lax.iota(jnp.int32, LANES)

    def chunk_body(c, carry):
        base = c * CHUNK
        for v in range(CHUNK // LANES):
            col = jlane + ((v * LANES) % COLS)
            sl = pl.ds(base + v * LANES, LANES)
            addr_v[sl] = addr_v[sl] * COLS + col
        return carry

    lax.fori_loop(0, NCHUNK, chunk_body, 0)

    pltpu.async_copy(in_hbm.at[addr_v], val_v, sem).wait()

    pltpu.sync_copy(val_v, out_hbm.at[w])


@jax.jit
def _gather_flat(flat_in, idx3):
    mesh = plsc.VectorSubcoreMesh(
        core_axis_name="c", subcore_axis_name="s",
        num_cores=NC, num_subcores=NS,
    )
    run = pl.kernel(
        _body,
        mesh=mesh,
        out_type=jax.ShapeDtypeStruct((NW, PER_W), jnp.float32),
        scratch_types=[
            pltpu.VMEM((PER_W,), jnp.int32),
            pltpu.VMEM((PER_W,), jnp.float32),
            pltpu.SemaphoreType.DMA,
        ],
    )
    return run(flat_in, idx3)


def kernel(input, index):
    flat_in = input.reshape(-1)
    idx3 = index.astype(jnp.int32).reshape(NW, PER_W)
    out = _gather_flat(flat_in, idx3)
    return out.reshape(ROWS, COLS)
